# baseline (speedup 1.0000x reference)
import functools

import jax
import jax.numpy as jnp
from jax import lax
from jax.experimental import pallas as pl
from jax.experimental.pallas import tpu as pltpu
from jax.experimental.pallas import tpu_sc as plsc

N_NODES = 10000
IN_DIM = 128
HID = 64
N_EDGES = 320000
N_PAIRS = 4096

NPAD = 10240
NC, NS, L = 2, 16, 16
CHUNK = 128
NCHUNKS = 2560
EPAD = NCHUNKS * CHUNK
CH_PER_TILE = NCHUNKS // NS
CH_PER_TILE_DEG = NCHUNKS // (NC * NS)
ROWS_PER_TILE = NPAD // NS

@functools.cache
def _mesh():
    return plsc.VectorSubcoreMesh(core_axis_name="c", subcore_axis_name="s",
                                  num_cores=NC, num_subcores=NS)


def _fill_zeros_2d(ref, nrows, ncols):
    z = jnp.zeros((L,), jnp.float32)
    for i in range(nrows):
        for jj in range(ncols // L):
            ref[i, pl.ds(jj * L, L)] = z


def _deg_body(ei3, degp, didx, ones, zb, degsh, dsem):
    c = lax.axis_index("c")
    s = lax.axis_index("s")
    w = c * NS + s

    one = jnp.ones((L,), jnp.float32)
    for i in range(CHUNK // L):
        ones[pl.ds(i * L, L)] = one
    z = jnp.zeros((L,), jnp.float32)
    for i in range(ROWS_PER_TILE // L):
        zb[pl.ds(i * L, L)] = z

    pltpu.sync_copy(zb, degsh.at[pl.ds(s * ROWS_PER_TILE, ROWS_PER_TILE)])
    plsc.subcore_barrier()

    pltpu.sync_copy(ei3.at[1, pl.ds(w * CH_PER_TILE_DEG, CH_PER_TILE_DEG)],
                    didx)

    def body(jo, carry):
        j = jo * 8
        for b in range(8):
            pltpu.async_copy(ones, degsh.at[didx.at[j + b]], dsem, add=True)
        for b in range(8):
            pltpu.make_async_copy(ones, degsh.at[didx.at[j + b]], dsem).wait()
        return carry

    lax.fori_loop(0, CH_PER_TILE_DEG // 8, body, 0)
    plsc.subcore_barrier()

    pltpu.sync_copy(degsh.at[pl.ds(s * ROWS_PER_TILE, ROWS_PER_TILE)],
                    degp.at[c, pl.ds(s * ROWS_PER_TILE, ROWS_PER_TILE)])


@functools.cache
def _deg_sc():
  return pl.kernel(
    _deg_body,
    out_type=jax.ShapeDtypeStruct((NC, NPAD), jnp.float32),
    mesh=_mesh(),
    scratch_types=[
        pltpu.VMEM((CH_PER_TILE_DEG, CHUNK), jnp.int32),
        pltpu.VMEM((CHUNK,), jnp.float32),
        pltpu.VMEM((ROWS_PER_TILE,), jnp.float32),
        pltpu.VMEM_SHARED((NPAD,), jnp.float32),
        pltpu.SemaphoreType.DMA,
    ],
)


def _pre_body(degp_ref, x_ref, xs_ref, dinv_ref):
    degp = degp_ref[...]
    deg = degp[0] + degp[1]
    dinv = lax.rsqrt(jnp.maximum(deg, 1.0))
    dcol = dinv[:, None]
    xs_ref[...] = jnp.concatenate(
        [x_ref[...] * dcol[:N_NODES],
         jnp.zeros((NPAD - N_NODES, IN_DIM), jnp.float32)], axis=0)
    dinv_ref[...] = dcol


def _pre_tc(degp, x):
    return pl.pallas_call(
        _pre_body,
        out_shape=(
            jax.ShapeDtypeStruct((NPAD, IN_DIM), jnp.float32),
            jax.ShapeDtypeStruct((NPAD, 1), jnp.float32),
        ),
    )(degp, x)


CH_PER_TILE_SC = NCHUNKS // (NC * NS)


GRP = CH_PER_TILE_SC // 2


def _scatter_body(ei3, xs, aggr, sidx, didx, rows0, rows1,
                  aggsh, gsem0, gsem1, ssem0, ssem1):
    c = lax.axis_index("c")
    s = lax.axis_index("s")

    _fill_zeros_2d(rows0, CHUNK, IN_DIM)
    base = (c * NS + s) * CH_PER_TILE_SC
    pltpu.sync_copy(ei3.at[0, pl.ds(base, GRP)], sidx)
    pltpu.sync_copy(ei3.at[1, pl.ds(base, GRP)], didx)
    for t in range(ROWS_PER_TILE // CHUNK):
        pltpu.sync_copy(rows0,
                        aggsh.at[pl.ds(s * ROWS_PER_TILE + t * CHUNK, CHUNK)])
    plsc.subcore_barrier()

    for g in range(2):
        gbase = base + g * GRP
        if g > 0:
            pltpu.sync_copy(ei3.at[0, pl.ds(gbase, GRP)], sidx)
            pltpu.sync_copy(ei3.at[1, pl.ds(gbase, GRP)], didx)

        pltpu.async_copy(xs.at[sidx.at[0]], rows0, gsem0)
        pltpu.async_copy(xs.at[sidx.at[1]], rows1, gsem1)

        def body(jo, carry):
            j = jo * 2
            pltpu.make_async_copy(xs.at[sidx.at[j]], rows0, gsem0).wait()
            pltpu.async_copy(rows0, aggsh.at[didx.at[j]], ssem0, add=True)
            pltpu.make_async_copy(xs.at[sidx.at[j + 1]], rows1, gsem1).wait()
            pltpu.async_copy(rows1, aggsh.at[didx.at[j + 1]], ssem1, add=True)
            pltpu.make_async_copy(rows0, aggsh.at[didx.at[j]], ssem0).wait()
            pltpu.async_copy(xs.at[sidx.at[j + 2]], rows0, gsem0)
            pltpu.make_async_copy(rows1, aggsh.at[didx.at[j + 1]], ssem1).wait()
            pltpu.async_copy(xs.at[sidx.at[j + 3]], rows1, gsem1)
            return carry

        lax.fori_loop(0, GRP // 2 - 1, body, 0)
        pltpu.make_async_copy(xs.at[sidx.at[GRP - 2]], rows0, gsem0).wait()
        pltpu.sync_copy(rows0, aggsh.at[didx.at[GRP - 2]], add=True)
        pltpu.make_async_copy(xs.at[sidx.at[GRP - 1]], rows1, gsem1).wait()
        pltpu.sync_copy(rows1, aggsh.at[didx.at[GRP - 1]], add=True)
    plsc.subcore_barrier()

    pltpu.sync_copy(aggsh.at[pl.ds(s * ROWS_PER_TILE, ROWS_PER_TILE)],
                    aggr.at[c, pl.ds(s * ROWS_PER_TILE, ROWS_PER_TILE)])


@functools.cache
def _scatter_sc():
  return pl.kernel(
    _scatter_body,
    out_type=jax.ShapeDtypeStruct((NC, NPAD, IN_DIM), jnp.float32),
    mesh=_mesh(),
    scratch_types=[
        pltpu.VMEM((GRP, CHUNK), jnp.int32),
        pltpu.VMEM((GRP, CHUNK), jnp.int32),
        pltpu.VMEM((CHUNK, IN_DIM), jnp.float32),
        pltpu.VMEM((CHUNK, IN_DIM), jnp.float32),
        pltpu.VMEM_SHARED((NPAD, IN_DIM), jnp.float32),
        pltpu.SemaphoreType.DMA,
        pltpu.SemaphoreType.DMA,
        pltpu.SemaphoreType.DMA,
        pltpu.SemaphoreType.DMA,
    ],
)


def _logsig(v):
    return jnp.minimum(v, 0.0) - jnp.log1p(jnp.exp(-jnp.abs(v)))


NB = 10
BROWS = N_NODES // NB


def _dense_body(aggr_ref, x_ref, dinv_ref, wl_ref, bl_ref,
                wh_ref, bh_ref, we_ref, be_ref, wd1_ref, bd1_ref,
                pq_ref, ssl_ref, acc_ref, zfirst_ref, zlast_ref):
    f32 = jnp.float32
    i = pl.program_id(0)

    @pl.when(i == 0)
    def _():
        acc_ref[...] = jnp.zeros((1, 1), f32)
        ssl_ref[...] = jnp.zeros((1, 1), f32)

    ap = aggr_ref[...]
    agg = (ap[0] + ap[1]) * dinv_ref[...]
    x = x_ref[...]
    low = jax.nn.relu(
        jnp.dot(agg, wl_ref[...], preferred_element_type=f32) + bl_ref[...])
    high = jax.nn.relu(
        jnp.dot(x - agg, wh_ref[...], preferred_element_type=f32)
        + bh_ref[...])
    we = we_ref[...]
    emb = jax.nn.relu(
        jnp.dot(low, we[:HID], preferred_element_type=f32)
        + jnp.dot(high, we[HID:], preferred_element_type=f32) + be_ref[...])

    wd1 = wd1_ref[...]
    p = (jnp.dot(emb, wd1[:HID], preferred_element_type=f32)
         + bd1_ref[...])
    q = jnp.dot(emb, wd1[HID:], preferred_element_type=f32)
    pq_ref[...] = jnp.concatenate([p, q], axis=1)

    nrm = jnp.sqrt(jnp.sum(emb * emb, axis=1, keepdims=True))
    z = emb / jnp.maximum(nrm, 1e-8)
    pos = jnp.concatenate([zlast_ref[...], z[:BROWS - 1]], axis=0)
    sim = jnp.sum(z * pos, axis=1, keepdims=True)
    ls = _logsig(sim)
    row0 = lax.broadcasted_iota(jnp.int32, (BROWS, 1), 0) == 0
    ls = jnp.where(jnp.logical_and(row0, i == 0), 0.0, ls)
    acc = acc_ref[...] + jnp.sum(ls)[None, None]
    acc_ref[...] = acc

    @pl.when(i == 0)
    def _():
        zfirst_ref[...] = z[0:1]

    zlast_ref[...] = z[BROWS - 1:BROWS]

    @pl.when(i == NB - 1)
    def _():
        sim0 = jnp.sum(zfirst_ref[...] * z[BROWS - 1:BROWS],
                       axis=1, keepdims=True)
        ssl_ref[...] = -(acc + _logsig(sim0)) / N_NODES


def _dense_tc(aggr, x, dinv, W_low, b_low, W_high, b_high,
              W_emb, b_emb, W_det1, b_det1):
    full = lambda shape: pl.BlockSpec(shape, lambda i: (0,) * len(shape))
    return pl.pallas_call(
        _dense_body,
        grid=(NB,),
        in_specs=[
            pl.BlockSpec((2, BROWS, IN_DIM), lambda i: (0, i, 0)),
            pl.BlockSpec((BROWS, IN_DIM), lambda i: (i, 0)),
            pl.BlockSpec((BROWS, 1), lambda i: (i, 0)),
            full((IN_DIM, HID)), full((1, HID)),
            full((IN_DIM, HID)), full((1, HID)),
            full((IN_DIM, HID)), full((1, HID)),
            full((IN_DIM, HID)), full((1, HID)),
        ],
        out_specs=(
            pl.BlockSpec((BROWS, IN_DIM), lambda i: (i, 0)),
            pl.BlockSpec((1, 1), lambda i: (0, 0)),
        ),
        out_shape=(
            jax.ShapeDtypeStruct((N_NODES, IN_DIM), jnp.float32),
            jax.ShapeDtypeStruct((1, 1), jnp.float32),
        ),
        scratch_shapes=[
            pltpu.VMEM((1, 1), jnp.float32),
            pltpu.VMEM((1, HID), jnp.float32),
            pltpu.VMEM((1, HID), jnp.float32),
        ],
    )(aggr, x, dinv, W_low, b_low.reshape(1, HID), W_high,
      b_high.reshape(1, HID), W_emb, b_emb.reshape(1, HID), W_det1,
      b_det1.reshape(1, HID))


def _pairs_body(pq_hbm, ai, bi, sout, idx, rowsa, rowsb, sbuf, sema, semb):
    c = lax.axis_index("c")
    s = lax.axis_index("s")
    w = c * NS + s
    base = w * CHUNK

    pltpu.sync_copy(ai.at[0, pl.ds(base, CHUNK)], idx.at[0])
    da = pltpu.async_copy(pq_hbm.at[idx.at[0]], rowsa, sema)
    pltpu.sync_copy(bi.at[0, pl.ds(base, CHUNK)], idx.at[1])
    db = pltpu.async_copy(pq_hbm.at[idx.at[1]], rowsb, semb)
    da.wait()
    db.wait()
    for i in range(CHUNK):
        for jj in range(HID // L):
            sbuf[i, pl.ds(jj * L, L)] = (
                rowsa[i, pl.ds(jj * L, L)]
                + rowsb[i, pl.ds(HID + jj * L, L)])
    pltpu.sync_copy(sbuf, sout.at[pl.ds(base, CHUNK)])


@functools.cache
def _pairs_sc():
  return pl.kernel(
    _pairs_body,
    out_type=jax.ShapeDtypeStruct((N_PAIRS, HID), jnp.float32),
    mesh=_mesh(),
    scratch_types=[
        pltpu.VMEM((2, CHUNK), jnp.int32),
        pltpu.VMEM((CHUNK, IN_DIM), jnp.float32),
        pltpu.VMEM((CHUNK, IN_DIM), jnp.float32),
        pltpu.VMEM((CHUNK, HID), jnp.float32),
        pltpu.SemaphoreType.DMA,
        pltpu.SemaphoreType.DMA,
    ],
)


def _head_body(s_ref, w2_ref, b2_ref, out_ref):
    h = jax.nn.relu(s_ref[...])
    t = jnp.dot(h, w2_ref[...], preferred_element_type=jnp.float32) \
        + b2_ref[...]
    out_ref[...] = jax.nn.sigmoid(t)


def _head_tc(s, W_det2, b_det2):
    return pl.pallas_call(
        _head_body,
        out_shape=jax.ShapeDtypeStruct((N_PAIRS, 1), jnp.float32),
    )(s, W_det2, b_det2.reshape(1, 1))


def kernel(x, edge_index, drugA_idx, drugB_idx, W_low, b_low, W_high, b_high,
           W_emb, b_emb, W_det1, b_det1, W_det2, b_det2):
    ei = edge_index.astype(jnp.int32)
    pad_tgt = N_NODES + jnp.arange(EPAD - N_EDGES, dtype=jnp.int32) \
        % (NPAD - N_NODES)
    pad = jnp.broadcast_to(pad_tgt, (2, EPAD - N_EDGES))
    ei3 = jnp.concatenate([ei, pad], axis=1).reshape(2, NCHUNKS, CHUNK)

    degp = _deg_sc()(ei3)
    xs, dinv = _pre_tc(degp, x)
    aggr = _scatter_sc()(ei3, xs)
    pq, ssl = _dense_tc(aggr, x, dinv, W_low, b_low, W_high,
                        b_high, W_emb, b_emb, W_det1, b_det1)
    sab = _pairs_sc()(pq, drugA_idx.astype(jnp.int32).reshape(1, N_PAIRS),
                    drugB_idx.astype(jnp.int32).reshape(1, N_PAIRS))
    pred = _head_tc(sab, W_det2, b_det2)
    return (pred, ssl[0, 0])


# speedup vs baseline: 1.0095x; 1.0095x over previous
import functools

import jax
import jax.numpy as jnp
from jax import lax
from jax.experimental import pallas as pl
from jax.experimental.pallas import tpu as pltpu
from jax.experimental.pallas import tpu_sc as plsc

N_NODES = 10000
IN_DIM = 128
HID = 64
N_EDGES = 320000
N_PAIRS = 4096

NPAD = 10240
NC, NS, L = 2, 16, 16
CHUNK = 128
NCHUNKS = 2560
EPAD = NCHUNKS * CHUNK
CH_PER_TILE = NCHUNKS // NS
CH_PER_TILE_DEG = NCHUNKS // (NC * NS)
ROWS_PER_TILE = NPAD // NS

@functools.cache
def _mesh():
    return plsc.VectorSubcoreMesh(core_axis_name="c", subcore_axis_name="s",
                                  num_cores=NC, num_subcores=NS)


def _fill_zeros_2d(ref, nrows, ncols):
    z = jnp.zeros((L,), jnp.float32)
    for i in range(nrows):
        for jj in range(ncols // L):
            ref[i, pl.ds(jj * L, L)] = z


def _deg_body(ei3, degp, didx, ones, zb, degsh, dsem):
    c = lax.axis_index("c")
    s = lax.axis_index("s")
    w = c * NS + s

    one = jnp.ones((L,), jnp.float32)
    for i in range(CHUNK // L):
        ones[pl.ds(i * L, L)] = one
    z = jnp.zeros((L,), jnp.float32)
    for i in range(ROWS_PER_TILE // L):
        zb[pl.ds(i * L, L)] = z

    pltpu.sync_copy(zb, degsh.at[pl.ds(s * ROWS_PER_TILE, ROWS_PER_TILE)])
    plsc.subcore_barrier()

    pltpu.sync_copy(ei3.at[1, pl.ds(w * CH_PER_TILE_DEG, CH_PER_TILE_DEG)],
                    didx)

    def body(jo, carry):
        j = jo * 8
        for b in range(8):
            pltpu.async_copy(ones, degsh.at[didx.at[j + b]], dsem, add=True)
        for b in range(8):
            pltpu.make_async_copy(ones, degsh.at[didx.at[j + b]], dsem).wait()
        return carry

    lax.fori_loop(0, CH_PER_TILE_DEG // 8, body, 0)
    plsc.subcore_barrier()

    pltpu.sync_copy(degsh.at[pl.ds(s * ROWS_PER_TILE, ROWS_PER_TILE)],
                    degp.at[c, pl.ds(s * ROWS_PER_TILE, ROWS_PER_TILE)])


@functools.cache
def _deg_sc():
  return pl.kernel(
    _deg_body,
    out_type=jax.ShapeDtypeStruct((NC, NPAD), jnp.float32),
    mesh=_mesh(),
    scratch_types=[
        pltpu.VMEM((CH_PER_TILE_DEG, CHUNK), jnp.int32),
        pltpu.VMEM((CHUNK,), jnp.float32),
        pltpu.VMEM((ROWS_PER_TILE,), jnp.float32),
        pltpu.VMEM_SHARED((NPAD,), jnp.float32),
        pltpu.SemaphoreType.DMA,
    ],
)


def _pre_body(degp_ref, x_ref, xs_ref, dinv_ref):
    degp = degp_ref[...]
    deg = degp[0] + degp[1]
    dinv = lax.rsqrt(jnp.maximum(deg, 1.0))
    dcol = dinv[:, None]
    xs_ref[...] = jnp.concatenate(
        [x_ref[...] * dcol[:N_NODES],
         jnp.zeros((NPAD - N_NODES, IN_DIM), jnp.float32)], axis=0)
    dinv_ref[...] = dcol


def _pre_tc(degp, x):
    return pl.pallas_call(
        _pre_body,
        out_shape=(
            jax.ShapeDtypeStruct((NPAD, IN_DIM), jnp.float32),
            jax.ShapeDtypeStruct((NPAD, 1), jnp.float32),
        ),
    )(degp, x)


CH_PER_TILE_SC = NCHUNKS // (NC * NS)


GRP = CH_PER_TILE_SC // 2


def _scatter_body(ei3, xs, aggr, sidx, didx, rows0, rows1,
                  aggsh, gsem0, gsem1, ssem0, ssem1):
    c = lax.axis_index("c")
    s = lax.axis_index("s")

    _fill_zeros_2d(rows0, CHUNK, IN_DIM)
    base = (c * NS + s) * CH_PER_TILE_SC
    pltpu.sync_copy(ei3.at[0, pl.ds(base, GRP)], sidx)
    pltpu.sync_copy(ei3.at[1, pl.ds(base, GRP)], didx)
    for t in range(ROWS_PER_TILE // CHUNK):
        pltpu.sync_copy(rows0,
                        aggsh.at[pl.ds(s * ROWS_PER_TILE + t * CHUNK, CHUNK)])
    plsc.subcore_barrier()

    for g in range(2):
        gbase = base + g * GRP
        if g > 0:
            pltpu.sync_copy(ei3.at[0, pl.ds(gbase, GRP)], sidx)
            pltpu.sync_copy(ei3.at[1, pl.ds(gbase, GRP)], didx)

        pltpu.async_copy(xs.at[sidx.at[0]], rows0, gsem0)
        pltpu.async_copy(xs.at[sidx.at[1]], rows1, gsem1)

        def body(jo, carry):
            j = jo * 2
            pltpu.make_async_copy(xs.at[sidx.at[j]], rows0, gsem0).wait()
            pltpu.async_copy(rows0, aggsh.at[didx.at[j]], ssem0, add=True)
            pltpu.make_async_copy(xs.at[sidx.at[j + 1]], rows1, gsem1).wait()
            pltpu.async_copy(rows1, aggsh.at[didx.at[j + 1]], ssem1, add=True)
            pltpu.make_async_copy(rows0, aggsh.at[didx.at[j]], ssem0).wait()
            pltpu.async_copy(xs.at[sidx.at[j + 2]], rows0, gsem0)
            pltpu.make_async_copy(rows1, aggsh.at[didx.at[j + 1]], ssem1).wait()
            pltpu.async_copy(xs.at[sidx.at[j + 3]], rows1, gsem1)
            return carry

        lax.fori_loop(0, GRP // 2 - 1, body, 0)
        pltpu.make_async_copy(xs.at[sidx.at[GRP - 2]], rows0, gsem0).wait()
        pltpu.sync_copy(rows0, aggsh.at[didx.at[GRP - 2]], add=True)
        pltpu.make_async_copy(xs.at[sidx.at[GRP - 1]], rows1, gsem1).wait()
        pltpu.sync_copy(rows1, aggsh.at[didx.at[GRP - 1]], add=True)
    plsc.subcore_barrier()

    pltpu.sync_copy(aggsh.at[pl.ds(s * ROWS_PER_TILE, ROWS_PER_TILE)],
                    aggr.at[c, pl.ds(s * ROWS_PER_TILE, ROWS_PER_TILE)])


@functools.cache
def _scatter_sc():
  return pl.kernel(
    _scatter_body,
    out_type=jax.ShapeDtypeStruct((NC, NPAD, IN_DIM), jnp.float32),
    mesh=_mesh(),
    scratch_types=[
        pltpu.VMEM((GRP, CHUNK), jnp.int32),
        pltpu.VMEM((GRP, CHUNK), jnp.int32),
        pltpu.VMEM((CHUNK, IN_DIM), jnp.float32),
        pltpu.VMEM((CHUNK, IN_DIM), jnp.float32),
        pltpu.VMEM_SHARED((NPAD, IN_DIM), jnp.float32),
        pltpu.SemaphoreType.DMA,
        pltpu.SemaphoreType.DMA,
        pltpu.SemaphoreType.DMA,
        pltpu.SemaphoreType.DMA,
    ],
)


def _dense_body(aggr_ref, x_ref, dinv_ref, wl_ref, bl_ref,
                wh_ref, bh_ref, we_ref, be_ref, wd1_ref, bd1_ref,
                pq_ref, ssl_ref):
    f32 = jnp.float32
    ap = aggr_ref[...]
    agg = (ap[0, :N_NODES] + ap[1, :N_NODES]) * dinv_ref[:N_NODES]
    x = x_ref[...]
    low = jax.nn.relu(
        jnp.dot(agg, wl_ref[...], preferred_element_type=f32) + bl_ref[...])
    high = jax.nn.relu(
        jnp.dot(x - agg, wh_ref[...], preferred_element_type=f32)
        + bh_ref[...])
    we = we_ref[...]
    emb = jax.nn.relu(
        jnp.dot(low, we[:HID], preferred_element_type=f32)
        + jnp.dot(high, we[HID:], preferred_element_type=f32) + be_ref[...])

    wd1 = wd1_ref[...]
    p = (jnp.dot(emb, wd1[:HID], preferred_element_type=f32)
         + bd1_ref[...])
    q = jnp.dot(emb, wd1[HID:], preferred_element_type=f32)
    pq_ref[...] = jnp.concatenate([p, q], axis=1)

    nrm = jnp.sqrt(jnp.sum(emb * emb, axis=1, keepdims=True))
    z = emb / jnp.maximum(nrm, 1e-8)
    pos = jnp.concatenate([z[N_NODES - 1:], z[:N_NODES - 1]], axis=0)
    sim = jnp.sum(z * pos, axis=1)
    logsig = jnp.minimum(sim, 0.0) - jnp.log1p(jnp.exp(-jnp.abs(sim)))
    ssl_ref[...] = (-jnp.mean(logsig))[None, None]


def _dense_tc(aggr, x, dinv, W_low, b_low, W_high, b_high,
              W_emb, b_emb, W_det1, b_det1):
    return pl.pallas_call(
        _dense_body,
        out_shape=(
            jax.ShapeDtypeStruct((N_NODES, IN_DIM), jnp.float32),
            jax.ShapeDtypeStruct((1, 1), jnp.float32),
        ),
    )(aggr, x, dinv, W_low, b_low.reshape(1, HID), W_high,
      b_high.reshape(1, HID), W_emb, b_emb.reshape(1, HID), W_det1,
      b_det1.reshape(1, HID))


def _pairs_body(pq_hbm, ai, bi, sout, idx, rowsa, rowsb, sbuf, sema, semb):
    c = lax.axis_index("c")
    s = lax.axis_index("s")
    w = c * NS + s
    base = w * CHUNK

    pltpu.sync_copy(ai.at[0, pl.ds(base, CHUNK)], idx.at[0])
    da = pltpu.async_copy(pq_hbm.at[idx.at[0]], rowsa, sema)
    pltpu.sync_copy(bi.at[0, pl.ds(base, CHUNK)], idx.at[1])
    db = pltpu.async_copy(pq_hbm.at[idx.at[1]], rowsb, semb)
    da.wait()
    db.wait()
    for i in range(CHUNK):
        for jj in range(HID // L):
            sbuf[i, pl.ds(jj * L, L)] = (
                rowsa[i, pl.ds(jj * L, L)]
                + rowsb[i, pl.ds(HID + jj * L, L)])
    pltpu.sync_copy(sbuf, sout.at[pl.ds(base, CHUNK)])


@functools.cache
def _pairs_sc():
  return pl.kernel(
    _pairs_body,
    out_type=jax.ShapeDtypeStruct((N_PAIRS, HID), jnp.float32),
    mesh=_mesh(),
    scratch_types=[
        pltpu.VMEM((2, CHUNK), jnp.int32),
        pltpu.VMEM((CHUNK, IN_DIM), jnp.float32),
        pltpu.VMEM((CHUNK, IN_DIM), jnp.float32),
        pltpu.VMEM((CHUNK, HID), jnp.float32),
        pltpu.SemaphoreType.DMA,
        pltpu.SemaphoreType.DMA,
    ],
)


def _head_body(s_ref, w2_ref, b2_ref, out_ref):
    h = jax.nn.relu(s_ref[...])
    t = jnp.dot(h, w2_ref[...], preferred_element_type=jnp.float32) \
        + b2_ref[...]
    out_ref[...] = jax.nn.sigmoid(t)


def _head_tc(s, W_det2, b_det2):
    return pl.pallas_call(
        _head_body,
        out_shape=jax.ShapeDtypeStruct((N_PAIRS, 1), jnp.float32),
    )(s, W_det2, b_det2.reshape(1, 1))


def kernel(x, edge_index, drugA_idx, drugB_idx, W_low, b_low, W_high, b_high,
           W_emb, b_emb, W_det1, b_det1, W_det2, b_det2):
    ei = edge_index.astype(jnp.int32)
    pad_tgt = N_NODES + jnp.arange(EPAD - N_EDGES, dtype=jnp.int32) \
        % (NPAD - N_NODES)
    pad = jnp.broadcast_to(pad_tgt, (2, EPAD - N_EDGES))
    ei3 = jnp.concatenate([ei, pad], axis=1).reshape(2, NCHUNKS, CHUNK)

    degp = _deg_sc()(ei3)
    xs, dinv = _pre_tc(degp, x)
    aggr = _scatter_sc()(ei3, xs)
    pq, ssl = _dense_tc(aggr, x, dinv, W_low, b_low, W_high,
                        b_high, W_emb, b_emb, W_det1, b_det1)
    sab = _pairs_sc()(pq, drugA_idx.astype(jnp.int32).reshape(1, N_PAIRS),
                    drugB_idx.astype(jnp.int32).reshape(1, N_PAIRS))
    pred = _head_tc(sab, W_det2, b_det2)
    return (pred, ssl[0, 0])
